# baseline (speedup 1.0000x reference)
import functools

import jax
import jax.numpy as jnp
from jax import lax
from jax.experimental import pallas as pl
from jax.experimental.pallas import tpu as pltpu
from jax.experimental.pallas import tpu_sc as plsc

DIM = 128
QUEUE_SIZE = 65536
BATCH_COLS = 4096

_SPLIT = 16384

_NW = 32
_COLS_W = _SPLIT // _NW
_CW = 256
_NCH = _COLS_W // _CW
_LK_WORKERS = BATCH_COLS // _COLS_W


def _sc_body(lk_ref, q_ref, out_ref, buf, rsem, wsem):
    cid = lax.axis_index("c")
    sid = lax.axis_index("s")
    wid = sid * 2 + cid
    col0 = wid * _COLS_W

    def ring(src_ref):
        for ch in range(_NCH):
            pltpu.make_async_copy(
                src_ref.at[:, pl.ds(col0 + ch * _CW, _CW)], buf.at[ch], rsem.at[ch]
            ).start()
        for ch in range(_NCH):
            pltpu.make_async_copy(
                src_ref.at[:, pl.ds(col0 + ch * _CW, _CW)], buf.at[ch], rsem.at[ch]
            ).wait()
            pltpu.make_async_copy(
                buf.at[ch], out_ref.at[:, pl.ds(col0 + ch * _CW, _CW)], wsem.at[ch]
            ).start()
        for ch in range(_NCH):
            pltpu.make_async_copy(
                buf.at[ch], out_ref.at[:, pl.ds(col0 + ch * _CW, _CW)], wsem.at[ch]
            ).wait()

    @pl.when(wid < _LK_WORKERS)
    def _():
        ring(lk_ref)

    @pl.when(wid >= _LK_WORKERS)
    def _():
        ring(q_ref)


def _sc_head(last_k, moco_queue):
    mesh = plsc.VectorSubcoreMesh(core_axis_name="c", subcore_axis_name="s")
    run = functools.partial(
        pl.kernel,
        out_type=jax.ShapeDtypeStruct((DIM, _SPLIT), jnp.float32),
        mesh=mesh,
        scratch_types=[
            pltpu.VMEM((_NCH, DIM, _CW), jnp.float32),
            pltpu.SemaphoreType.DMA((_NCH,)),
            pltpu.SemaphoreType.DMA((_NCH,)),
        ],
    )(_sc_body)
    return run(last_k, moco_queue)


_TW = 8192
_TCHUNK = (QUEUE_SIZE - _SPLIT) // _TW
_TNBUF = 4


def _tc_body(q_ref, out_ref, buf, rsem, wsem):
    def rd(c):
        b = c % _TNBUF
        return pltpu.make_async_copy(
            q_ref.at[:, pl.ds(_SPLIT + c * _TW, _TW)], buf.at[b], rsem.at[b]
        )

    def wr(c):
        b = c % _TNBUF
        return pltpu.make_async_copy(
            buf.at[b], out_ref.at[:, pl.ds(c * _TW, _TW)], wsem.at[b]
        )

    for c in range(_TNBUF):
        rd(c).start()
    for c in range(_TCHUNK):
        rd(c).wait()
        wr(c).start()
        if c + _TNBUF < _TCHUNK:
            wr(c).wait()
            rd(c + _TNBUF).start()
    for c in range(max(_TCHUNK - _TNBUF, 0), _TCHUNK):
        wr(c).wait()


def _tc_tail(moco_queue):
    return pl.pallas_call(
        _tc_body,
        in_specs=[pl.BlockSpec(memory_space=pl.ANY)],
        out_specs=pl.BlockSpec(memory_space=pl.ANY),
        out_shape=jax.ShapeDtypeStruct((DIM, QUEUE_SIZE - _SPLIT), jnp.float32),
        scratch_shapes=[
            pltpu.VMEM((_TNBUF, DIM, _TW), jnp.float32),
            pltpu.SemaphoreType.DMA((_TNBUF,)),
            pltpu.SemaphoreType.DMA((_TNBUF,)),
        ],
    )(moco_queue)


def kernel(last_k, moco_queue):
    head = _sc_head(last_k, moco_queue)
    tail = _tc_tail(moco_queue)
    return jnp.concatenate([head, tail], axis=1)


# speedup vs baseline: 1.4292x; 1.4292x over previous
import functools

import jax
import jax.numpy as jnp
from jax import lax
from jax.experimental import pallas as pl
from jax.experimental.pallas import tpu as pltpu
from jax.experimental.pallas import tpu_sc as plsc

DIM = 128
QUEUE_SIZE = 65536
BATCH_COLS = 4096

_NW = 32
_ROWS_W = 8
_HALF = QUEUE_SIZE // 2
_CW = 4096
_NCH = _HALF // _CW
_NBUF = 3


def _sc_body(lk_ref, q_ref, out_ref, buf, rsem, wsem):
    cid = lax.axis_index("c")
    sid = lax.axis_index("s")
    wid = sid * 2 + cid
    r = wid // 2
    h = wid % 2
    row0 = r * _ROWS_W
    colbase = h * _HALF

    def rd(ch):
        b = ch % _NBUF
        return pltpu.make_async_copy(
            q_ref.at[pl.ds(row0, _ROWS_W), pl.ds(colbase + ch * _CW, _CW)],
            buf.at[b],
            rsem.at[b],
        )

    def rd_lk(b):
        return pltpu.make_async_copy(
            lk_ref.at[pl.ds(row0, _ROWS_W), :], buf.at[b], rsem.at[b]
        )

    def wr(ch):
        b = ch % _NBUF
        return pltpu.make_async_copy(
            buf.at[b],
            out_ref.at[pl.ds(row0, _ROWS_W), pl.ds(colbase + ch * _CW, _CW)],
            wsem.at[b],
        )

    def start_read(ch):
        if ch == 0:
            @pl.when(h == 0)
            def _():
                rd_lk(0).start()

            @pl.when(h != 0)
            def _():
                rd(0).start()
        else:
            rd(ch).start()

    for ch in range(_NBUF):
        start_read(ch)
    for ch in range(_NCH):
        rd(ch).wait()
        wr(ch).start()
        if ch + _NBUF < _NCH:
            wr(ch).wait()
            start_read(ch + _NBUF)
    for ch in range(max(_NCH - _NBUF, 0), _NCH):
        wr(ch).wait()


def kernel(last_k, moco_queue):
    mesh = plsc.VectorSubcoreMesh(core_axis_name="c", subcore_axis_name="s")
    run = functools.partial(
        pl.kernel,
        out_type=jax.ShapeDtypeStruct((DIM, QUEUE_SIZE), jnp.float32),
        mesh=mesh,
        scratch_types=[
            pltpu.VMEM((_NBUF, _ROWS_W, _CW), jnp.float32),
            pltpu.SemaphoreType.DMA((_NBUF,)),
            pltpu.SemaphoreType.DMA((_NBUF,)),
        ],
    )(_sc_body)
    return run(last_k, moco_queue)


# speedup vs baseline: 2.8835x; 2.0176x over previous
import jax
import jax.numpy as jnp
from jax.experimental import pallas as pl
from jax.experimental.pallas import tpu as pltpu

DIM = 128
QUEUE_SIZE = 65536
BATCH_COLS = 4096

_W = 16384
_NCHUNK = QUEUE_SIZE // _W
_NBUF = 3


def _ring_body(lk_ref, q_ref, out_ref, buf, rsem, wsem):
    def read_descs(c):
        b = c % _NBUF
        if c == 0:
            return [
                pltpu.make_async_copy(lk_ref, buf.at[b, :, pl.ds(0, BATCH_COLS)], rsem.at[b]),
                pltpu.make_async_copy(
                    q_ref.at[:, pl.ds(BATCH_COLS, _W - BATCH_COLS)],
                    buf.at[b, :, pl.ds(BATCH_COLS, _W - BATCH_COLS)],
                    rsem.at[b],
                ),
            ]
        return [
            pltpu.make_async_copy(
                q_ref.at[:, pl.ds(c * _W, _W)], buf.at[b], rsem.at[b]
            )
        ]

    def write_desc(c):
        b = c % _NBUF
        return pltpu.make_async_copy(
            buf.at[b], out_ref.at[:, pl.ds(c * _W, _W)], wsem.at[b]
        )

    for c in range(_NBUF):
        for d in read_descs(c):
            d.start()
    for c in range(_NCHUNK):
        for d in read_descs(c):
            d.wait()
        write_desc(c).start()
        if c + _NBUF < _NCHUNK:
            write_desc(c).wait()
            for d in read_descs(c + _NBUF):
                d.start()
    for c in range(max(_NCHUNK - _NBUF, 0), _NCHUNK):
        write_desc(c).wait()


def kernel(last_k, moco_queue):
    return pl.pallas_call(
        _ring_body,
        in_specs=[
            pl.BlockSpec(memory_space=pl.ANY),
            pl.BlockSpec(memory_space=pl.ANY),
        ],
        out_specs=pl.BlockSpec(memory_space=pl.ANY),
        out_shape=jax.ShapeDtypeStruct((DIM, QUEUE_SIZE), jnp.float32),
        scratch_shapes=[
            pltpu.VMEM((_NBUF, DIM, _W), jnp.float32),
            pltpu.SemaphoreType.DMA((_NBUF,)),
            pltpu.SemaphoreType.DMA((_NBUF,)),
        ],
    )(last_k, moco_queue)
